# Initial kernel scaffold; baseline (speedup 1.0000x reference)
import jax
import jax.numpy as jnp
from jax.experimental import pallas as pl


def kernel(x, edge_index, W1, b1, W2, b2):
    raise NotImplementedError("write your pallas kernel here")


# speedup vs baseline: 3.4322x; 3.4322x over previous
import functools

import jax
import jax.numpy as jnp
from jax import lax
from jax.experimental import pallas as pl
from jax.experimental.pallas import tpu as pltpu
from jax.experimental.pallas import tpu_sc as plsc

N = 10000
E = 160000
F = 256
HALF = 128

N_PAD = 10240
DUMMY = N

NUM_TILES = 16
E_PER_TILE = E // NUM_TILES
CHUNK = 80
N_CHUNKS = E_PER_TILE // CHUNK
ZROWS = 16
ROWS_OUT = 632
ROWS_OUT_LAST = N - 15 * ROWS_OUT


def _mm1_body(x_ref, w_ref, b_ref, o_ref):
    acc = lax.dot_general(
        x_ref[...], w_ref[...], (((1,), (1,)), ((), ())),
        preferred_element_type=jnp.float32)
    o_ref[...] = acc + b_ref[...]


def _mm1(x, W1, b1):
    blk = 400
    grid = (N // blk, 2)
    return pl.pallas_call(
        _mm1_body,
        grid=grid,
        in_specs=[
            pl.BlockSpec((blk, F), lambda i, c: (i, 0)),
            pl.BlockSpec((HALF, F), lambda i, c: (c, 0)),
            pl.BlockSpec((1, HALF), lambda i, c: (0, c)),
        ],
        out_specs=pl.BlockSpec((blk, HALF), lambda i, c: (c * (N // blk) + i, 0)),
        out_shape=jax.ShapeDtypeStruct((2 * N, HALF), jnp.float32),
    )(x, W1, b1.reshape(1, F))


def _sc_body(h_hbm, src_hbm, dst_hbm, out_hbm,
             src_buf, dst_buf, rows_v, zbuf, agg_sh, sem):
    cid = lax.axis_index("c")
    sid = lax.axis_index("s")

    for r in range(ZROWS):
        for c in range(HALF // 16):
            zbuf[r, pl.ds(c * 16, 16)] = jnp.zeros((16,), jnp.float32)

    zband = N_PAD // NUM_TILES
    zbase = sid * zband

    def _zero(j, carry):
        pltpu.sync_copy(zbuf, agg_sh.at[pl.ds(zbase + j * ZROWS, ZROWS)])
        return carry

    lax.fori_loop(0, zband // ZROWS, _zero, 0)
    plsc.subcore_barrier()

    ebase = sid * E_PER_TILE
    off = cid * N

    def _chunk(j, carry):
        start = ebase + j * CHUNK
        pltpu.sync_copy(src_hbm.at[pl.ds(start, CHUNK)], src_buf)
        pltpu.sync_copy(dst_hbm.at[pl.ds(start, CHUNK)], dst_buf)

        def _adj(i, c2):
            s = src_buf[pl.ds(i * 16, 16)]
            d = dst_buf[pl.ds(i * 16, 16)]
            src_buf[pl.ds(i * 16, 16)] = s + off
            dst_buf[pl.ds(i * 16, 16)] = jnp.where(s == d, DUMMY, d)
            return c2

        lax.fori_loop(0, CHUNK // 16, _adj, 0)
        pltpu.async_copy(h_hbm.at[src_buf], rows_v, sem).wait()
        pltpu.sync_copy(rows_v, agg_sh.at[dst_buf], add=True)
        return carry

    lax.fori_loop(0, N_CHUNKS, _chunk, 0)
    plsc.subcore_barrier()

    @pl.when(sid < NUM_TILES - 1)
    def _copy_main():
        pltpu.sync_copy(agg_sh.at[pl.ds(sid * ROWS_OUT, ROWS_OUT)],
                        out_hbm.at[pl.ds(cid * N + sid * ROWS_OUT, ROWS_OUT)])

    @pl.when(sid == NUM_TILES - 1)
    def _copy_last():
        base = (NUM_TILES - 1) * ROWS_OUT
        pltpu.sync_copy(agg_sh.at[pl.ds(base, ROWS_OUT_LAST)],
                        out_hbm.at[pl.ds(cid * N + base, ROWS_OUT_LAST)])


def _sc_scatter(h2, src, dst):
    mesh = plsc.VectorSubcoreMesh(core_axis_name="c", subcore_axis_name="s")
    k = functools.partial(
        pl.kernel,
        mesh=mesh,
        out_type=jax.ShapeDtypeStruct((2 * N, HALF), jnp.float32),
        scratch_types=[
            pltpu.VMEM((CHUNK,), jnp.int32),
            pltpu.VMEM((CHUNK,), jnp.int32),
            pltpu.VMEM((CHUNK, HALF), jnp.float32),
            pltpu.VMEM((ZROWS, HALF), jnp.float32),
            pltpu.VMEM_SHARED((N_PAD, HALF), jnp.float32),
            pltpu.SemaphoreType.DMA,
        ],
    )(_sc_body)
    return k(h2, src, dst)


def _mm2_body(a0_ref, a1_ref, w_ref, b_ref, o_ref):
    r0 = jnp.maximum(a0_ref[...], 0.0)
    r1 = jnp.maximum(a1_ref[...], 0.0)
    acc = lax.dot_general(
        r0, w_ref[:, :HALF], (((1,), (1,)), ((), ())),
        preferred_element_type=jnp.float32)
    acc += lax.dot_general(
        r1, w_ref[:, HALF:], (((1,), (1,)), ((), ())),
        preferred_element_type=jnp.float32)
    o_ref[...] = acc + b_ref[...]


def _mm2(agg2, W2, b2):
    blk = 400
    grid = (N // blk,)
    return pl.pallas_call(
        _mm2_body,
        grid=grid,
        in_specs=[
            pl.BlockSpec((blk, HALF), lambda i: (i, 0)),
            pl.BlockSpec((blk, HALF), lambda i: ((N // blk) + i, 0)),
            pl.BlockSpec((F, F), lambda i: (0, 0)),
            pl.BlockSpec((1, F), lambda i: (0, 0)),
        ],
        out_specs=pl.BlockSpec((blk, F), lambda i: (i, 0)),
        out_shape=jax.ShapeDtypeStruct((N, F), jnp.float32),
    )(agg2, agg2, W2, b2.reshape(1, F))


def kernel(x, edge_index, W1, b1, W2, b2):
    h2 = _mm1(x, W1, b1)
    agg2 = _sc_scatter(h2, edge_index[0], edge_index[1])
    return _mm2(agg2, W2, b2)
